# Initial kernel scaffold; baseline (speedup 1.0000x reference)
import jax
import jax.numpy as jnp
from jax.experimental import pallas as pl


def kernel(x, router_w, router_b, w_c_fc, b_c_fc, w_gate, b_gate, w_c_proj, b_c_proj):
    raise NotImplementedError("write your pallas kernel here")


# speedup vs baseline: 1.7524x; 1.7524x over previous
import functools

import jax
import jax.numpy as jnp
from jax import lax
from jax.experimental import pallas as pl
from jax.experimental.pallas import tpu as pltpu
from jax.experimental.pallas import tpu_sc as plsc

E = 8
K = 2
EPAD = 128
NC = 2
NS = 16
NW = NC * NS


def _routing_body(Tt, Bb, cap, R, x_ref, rw_ref, rb_ref, slots_ref, probs_ref,
                  ex0_ref, ex1_ref, oh0_ref, oh1_ref):
    BT = Bb * Tt
    logits = jnp.dot(x_ref[...], rw_ref[...],
                     preferred_element_type=jnp.float32) + rb_ref[...]
    col = lax.broadcasted_iota(jnp.int32, (BT, EPAD), 1)

    m1 = jnp.max(logits, axis=1, keepdims=True)
    e1 = jnp.min(jnp.where(logits == m1, col, EPAD), axis=1, keepdims=True)
    neg = jnp.where(col == e1, -jnp.inf, logits)
    m2 = jnp.max(neg, axis=1, keepdims=True)
    e2 = jnp.min(jnp.where(neg == m2, col, EPAD), axis=1, keepdims=True)

    p1 = 1.0 / (1.0 + jnp.exp(m2 - m1))
    p2 = 1.0 - p1

    oh0 = (col == e1).astype(jnp.float32)
    oh1 = (col == e2).astype(jnp.float32)
    oh0_ref[...] = oh0
    oh1_ref[...] = oh1

    r128 = lax.broadcasted_iota(jnp.int32, (128, 128), 0)
    c128 = lax.broadcasted_iota(jnp.int32, (128, 128), 1)
    ltri = (c128 < r128).astype(jnp.float32)
    nblk = BT // 128
    blk_per_batch = Tt // 128

    def blk_step(j, carries):
        c0, c1 = carries
        fresh = (j % blk_per_batch) == 0
        c0 = jnp.where(fresh, jnp.zeros_like(c0), c0)
        c1 = jnp.where(fresh, jnp.zeros_like(c1), c1)
        b0 = oh0_ref[pl.ds(j * 128, 128), :]
        b1 = oh1_ref[pl.ds(j * 128, 128), :]
        ex0_ref[pl.ds(j * 128, 128), :] = (
            jnp.dot(ltri, b0, preferred_element_type=jnp.float32) + c0)
        ex1_ref[pl.ds(j * 128, 128), :] = (
            jnp.dot(ltri, b1, preferred_element_type=jnp.float32) + c1)
        c0 = c0 + jnp.sum(b0, axis=0, keepdims=True)
        c1 = c1 + jnp.sum(b1, axis=0, keepdims=True)
        return (c0, c1)

    zero_c = jnp.zeros((1, EPAD), jnp.float32)
    lax.fori_loop(0, nblk, blk_step, (zero_c, zero_c))

    row = lax.broadcasted_iota(jnp.int32, (BT, 1), 0)
    bvec = (row // Tt).astype(jnp.float32)
    s0_per_batch = []
    for b in range(Bb):
        s0_per_batch.append(
            jnp.sum(oh0[b * Tt:(b + 1) * Tt, :], axis=0, keepdims=True))
    s0_bcast = jnp.zeros((BT, EPAD), jnp.float32)
    for b in range(Bb):
        sel = (row == row) & ((row // Tt) == b)
        s0_bcast = s0_bcast + jnp.where(sel, s0_per_batch[b], 0.0)

    pos0 = jnp.sum(oh0 * ex0_ref[...], axis=1, keepdims=True)
    pos1 = jnp.sum(oh1 * (ex1_ref[...] + s0_bcast), axis=1, keepdims=True)

    e1f = e1.astype(jnp.float32)
    e2f = e2.astype(jnp.float32)
    base0 = e1f * R + bvec * cap
    base1 = e2f * R + bvec * cap
    trash0 = e1f * R + Bb * cap
    trash1 = e2f * R + Bb * cap
    dst0 = jnp.where(pos0 < cap, base0 + pos0, trash0)
    dst1 = jnp.where(pos1 < cap, base1 + pos1, trash1)
    comb0 = base0 + jnp.minimum(pos0, cap - 1.0)
    comb1 = base1 + jnp.minimum(pos1, cap - 1.0)

    ocol = lax.broadcasted_iota(jnp.int32, (BT, E), 1)
    slots = (jnp.where(ocol == 0, dst0, 0.0) + jnp.where(ocol == 1, dst1, 0.0)
             + jnp.where(ocol == 2, comb0, 0.0)
             + jnp.where(ocol == 3, comb1, 0.0))
    slots_ref[...] = slots.astype(jnp.int32)
    probs_ref[...] = (jnp.where(ocol == 0, p1, 0.0)
                      + jnp.where(ocol == 1, p2, 0.0))


def _make_routing(Tt, Bb, cap, R, interpret=False):
    BT = Bb * Tt
    return pl.pallas_call(
        functools.partial(_routing_body, Tt, Bb, cap, R),
        grid=(1,),
        in_specs=[
            pl.BlockSpec((BT, 1024), lambda i: (0, 0)),
            pl.BlockSpec((1024, EPAD), lambda i: (0, 0)),
            pl.BlockSpec((1, EPAD), lambda i: (0, 0)),
        ],
        out_specs=[
            pl.BlockSpec((BT, E), lambda i: (0, 0)),
            pl.BlockSpec((BT, E), lambda i: (0, 0)),
        ],
        out_shape=[
            jax.ShapeDtypeStruct((BT, E), jnp.int32),
            jax.ShapeDtypeStruct((BT, E), jnp.float32),
        ],
        scratch_shapes=[
            pltpu.VMEM((BT, EPAD), jnp.float32),
            pltpu.VMEM((BT, EPAD), jnp.float32),
            pltpu.VMEM((BT, EPAD), jnp.float32),
            pltpu.VMEM((BT, EPAD), jnp.float32),
        ],
        interpret=interpret,
    )


def _make_dispatch(BT, C, nrows):
    chunk = 64
    n_chunks = BT // (NW * chunk)
    mesh = plsc.VectorSubcoreMesh(core_axis_name="c", subcore_axis_name="s")

    @functools.partial(
        pl.kernel,
        out_type=jax.ShapeDtypeStruct((nrows, C), jnp.float32),
        mesh=mesh,
        scratch_types=[
            pltpu.VMEM((chunk, C), jnp.float32),
            pltpu.VMEM((chunk,), jnp.int32),
            pltpu.VMEM((chunk,), jnp.int32),
            pltpu.SemaphoreType.DMA,
            pltpu.SemaphoreType.DMA,
        ],
    )
    def dispatch(x_hbm, dst0_hbm, dst1_hbm, buf_hbm, rows_v, idx0_v, idx1_v,
                 sem0, sem1):
        wid = lax.axis_index("s") * NC + lax.axis_index("c")
        for ci in range(n_chunks):
            base = wid * (n_chunks * chunk) + ci * chunk
            pltpu.sync_copy(x_hbm.at[pl.ds(base, chunk)], rows_v)
            pltpu.sync_copy(dst0_hbm.at[pl.ds(base, chunk)], idx0_v)
            pltpu.sync_copy(dst1_hbm.at[pl.ds(base, chunk)], idx1_v)
            c0 = pltpu.async_copy(rows_v, buf_hbm.at[idx0_v], sem0)
            c1 = pltpu.async_copy(rows_v, buf_hbm.at[idx1_v], sem1)
            c0.wait()
            c1.wait()

    return dispatch


def _ffn_body(R, HB, NH, buf_ref, w1_ref, b1_ref, wg_ref, bg_ref, w2_ref,
              b2_ref, out_ref):
    h = pl.program_id(1)

    def m_step(m, _):
        xt = buf_ref[pl.ds(m * 128, 128), :]
        hh = jnp.dot(xt, w1_ref[0], preferred_element_type=jnp.float32)
        hh = hh + b1_ref[0]
        gg = jnp.dot(xt, wg_ref[0], preferred_element_type=jnp.float32)
        gg = gg + bg_ref[0]
        z = hh * gg
        a = z * jax.nn.sigmoid(z)
        part = jnp.dot(a, w2_ref[0], preferred_element_type=jnp.float32)

        @pl.when(h == 0)
        def _():
            out_ref[pl.ds(m * 128, 128), :] = part

        @pl.when(h > 0)
        def _():
            out_ref[pl.ds(m * 128, 128), :] = (
                out_ref[pl.ds(m * 128, 128), :] + part)

        return 0

    lax.fori_loop(0, R // 128, m_step, 0)

    @pl.when(h == NH - 1)
    def _():
        out_ref[...] = out_ref[...] + b2_ref[0]


def _make_ffn(R, C, H, interpret=False):
    HB = 512
    NH = H // HB
    return pl.pallas_call(
        functools.partial(_ffn_body, R, HB, NH),
        grid=(E, NH),
        in_specs=[
            pl.BlockSpec((R, C), lambda e, h: (e, 0)),
            pl.BlockSpec((1, C, HB), lambda e, h: (e, 0, h)),
            pl.BlockSpec((1, 1, HB), lambda e, h: (e, 0, h)),
            pl.BlockSpec((1, C, HB), lambda e, h: (e, 0, h)),
            pl.BlockSpec((1, 1, HB), lambda e, h: (e, 0, h)),
            pl.BlockSpec((1, HB, C), lambda e, h: (e, h, 0)),
            pl.BlockSpec((1, 1, C), lambda e, h: (e, 0, 0)),
        ],
        out_specs=pl.BlockSpec((R, C), lambda e, h: (e, 0)),
        out_shape=jax.ShapeDtypeStruct((E * R, C), jnp.float32),
        compiler_params=pltpu.CompilerParams(
            dimension_semantics=("parallel", "arbitrary")),
        interpret=interpret,
    )


def _make_combine(BT, C, nrows):
    chunk = 32
    n_chunks = BT // (NW * chunk)
    lanes_per_row = C // 16
    mesh = plsc.VectorSubcoreMesh(core_axis_name="c", subcore_axis_name="s")

    @functools.partial(
        pl.kernel,
        out_type=jax.ShapeDtypeStruct((BT, C), jnp.float32),
        mesh=mesh,
        scratch_types=[
            pltpu.VMEM((chunk, C), jnp.float32),
            pltpu.VMEM((chunk, C), jnp.float32),
            pltpu.VMEM((chunk,), jnp.int32),
            pltpu.VMEM((chunk,), jnp.int32),
            pltpu.VMEM((chunk,), jnp.float32),
            pltpu.VMEM((chunk,), jnp.float32),
            pltpu.SemaphoreType.DMA,
            pltpu.SemaphoreType.DMA,
        ],
        compiler_params=pltpu.CompilerParams(needs_layout_passes=False),
    )
    def combine(ffn_hbm, comb0_hbm, comb1_hbm, p0_hbm, p1_hbm, y_hbm,
                r0_v, r1_v, idx0_v, idx1_v, p0_v, p1_v, sem0, sem1):
        wid = lax.axis_index("s") * NC + lax.axis_index("c")
        for ci in range(n_chunks):
            base = wid * (n_chunks * chunk) + ci * chunk
            pltpu.sync_copy(comb0_hbm.at[pl.ds(base, chunk)], idx0_v)
            pltpu.sync_copy(comb1_hbm.at[pl.ds(base, chunk)], idx1_v)
            pltpu.sync_copy(p0_hbm.at[pl.ds(base, chunk)], p0_v)
            pltpu.sync_copy(p1_hbm.at[pl.ds(base, chunk)], p1_v)
            c0 = pltpu.async_copy(ffn_hbm.at[idx0_v], r0_v, sem0)
            c1 = pltpu.async_copy(ffn_hbm.at[idx1_v], r1_v, sem1)
            c0.wait()
            c1.wait()

            def row_step(r, _):
                ridx = jnp.full((16,), r, dtype=jnp.int32)
                p0s = plsc.load_gather(p0_v, [ridx])
                p1s = plsc.load_gather(p1_v, [ridx])
                for c in range(lanes_per_row):
                    a = r0_v[r, pl.ds(c * 16, 16)]
                    b = r1_v[r, pl.ds(c * 16, 16)]
                    r0_v[r, pl.ds(c * 16, 16)] = p0s * a + p1s * b
                return 0

            lax.fori_loop(0, chunk, row_step, 0)
            pltpu.sync_copy(r0_v, y_hbm.at[pl.ds(base, chunk)])

    return combine


def kernel(x, router_w, router_b, w_c_fc, b_c_fc, w_gate, b_gate, w_c_proj,
           b_c_proj):
    Bb, Tt, C = x.shape
    H = w_c_fc.shape[-1]
    cap = int(1.2 * K * Tt // E)
    R = ((Bb * cap + 1 + 127) // 128) * 128
    BT = Bb * Tt

    xf = x.reshape(BT, C)
    rw_pad = jnp.zeros((C, EPAD), jnp.float32).at[:, :E].set(router_w)
    rb_pad = jnp.full((1, EPAD), -1e30, jnp.float32).at[0, :E].set(router_b)

    slots, probs = _make_routing(Tt, Bb, cap, R)(xf, rw_pad, rb_pad)
    dst0 = slots[:, 0]
    dst1 = slots[:, 1]
    comb0 = slots[:, 2]
    comb1 = slots[:, 3]
    p0 = probs[:, 0]
    p1 = probs[:, 1]

    buf = _make_dispatch(BT, C, E * R)(xf, dst0, dst1)
    ffn = _make_ffn(R, C, H)(buf, w_c_fc, b_c_fc, w_gate, b_gate, w_c_proj,
                             b_c_proj)
    y = _make_combine(BT, C, E * R)(ffn, comb0, comb1, p0, p1)
    return y.reshape(Bb, Tt, C)
